# Initial kernel scaffold; baseline (speedup 1.0000x reference)
import jax
import jax.numpy as jnp
from jax.experimental import pallas as pl


def kernel(nfeatures, efeatures, next_flowrate, weights, edge_index, inlet_mask):
    raise NotImplementedError("write your pallas kernel here")


# speedup vs baseline: 2.2718x; 2.2718x over previous
import functools

import jax
import jax.numpy as jnp
from jax import lax
from jax.experimental import pallas as pl
from jax.experimental.pallas import tpu as pltpu
from jax.experimental.pallas import tpu_sc as plsc

N = 100000
E = 1600000

CHUNK = 128
PT_CH = 400
PT_E = PT_CH * CHUNK
NTILES = 32
EPAD = NTILES * PT_E
NCH_TOT = EPAD // CHUNK

GSZ_G = 16
GROUPS_G = PT_CH // GSZ_G
GRP_EG = GSZ_G * CHUNK
GSZ_S = 8
GROUPS_S = PT_CH // GSZ_S
GRP_ES = GSZ_S * CHUNK

N_ACC = 100352
ZROWS = N_ACC // 16
ZCH = ZROWS // CHUNK
RD = 6256
RD_LAST = N - 15 * RD

RN = 2000
RE = 4096


def _lrelu(x):
    return jnp.where(x >= 0, x, 0.01 * x)


def _ln(f, g, b):
    mu = jnp.mean(f, axis=-1, keepdims=True)
    d = f - mu
    var = jnp.mean(d * d, axis=-1, keepdims=True)
    return d / jnp.sqrt(var + 1e-5) * g + b


def _dot(x, w):
    return jnp.dot(x.astype(jnp.bfloat16), w.astype(jnp.bfloat16),
                   preferred_element_type=jnp.float32)


def _mlp_tail(h, w1, b1, w2, b2, wo, bo):
    h = _lrelu(_dot(h, w1) + b1)
    h = _lrelu(_dot(h, w2) + b2)
    return _dot(h, wo) + bo


def _wspec(w):
    nd = w.ndim
    return pl.BlockSpec(w.shape, lambda i, _nd=nd: (0,) * _nd)


def _flat_mlp(w, norm):
    (w1, b1), (w2, b2) = w['hidden']
    out = [w['Wi'], w['bi'].reshape(1, -1), w1, b1.reshape(1, -1),
           w2, b2.reshape(1, -1), w['Wo'], w['bo'].reshape(1, -1)]
    if norm:
        out += [w['g'].reshape(1, -1), w['b'].reshape(1, -1)]
    return out


def _enc_n_body(x_ref, fl_ref, mk_ref, wi, bi, w1, b1, w2, b2, wo, bo, g, b,
                out_ref):
    x = x_ref[...]
    nf = jnp.where(mk_ref[...] != 0, fl_ref[...], 0.0)
    nf = nf.astype(jnp.bfloat16).astype(jnp.float32)
    W = wi[...]
    w11 = W[11:12].astype(jnp.bfloat16).astype(jnp.float32)
    h = _lrelu(_dot(x, W[0:11]) + nf * w11 + bi[...])
    f = _mlp_tail(h, w1[...], b1[...], w2[...], b2[...], wo[...], bo[...])
    out_ref[...] = _ln(f, g[...], b[...])


def _enc_e_body(x_ref, wi, bi, w1, b1, w2, b2, wo, bo, g, b, out_ref):
    h = _lrelu(_dot(x_ref[...], wi[...]) + bi[...])
    f = _mlp_tail(h, w1[...], b1[...], w2[...], b2[...], wo[...], bo[...])
    out_ref[...] = _ln(f, g[...], b[...])


def _edge_mlp_body(pe_ref, gs_ref, gd_ref, wi, bi, w1, b1, w2, b2, wo, bo,
                   g, b, out_ref):
    pe = pe_ref[...]
    W = wi[...]
    h = _lrelu(_dot(pe, W[0:16]) + _dot(gs_ref[...], W[16:32])
               + _dot(gd_ref[...], W[32:48]) + bi[...])
    f = _mlp_tail(h, w1[...], b1[...], w2[...], b2[...], wo[...], bo[...])
    out_ref[...] = _ln(f, g[...], b[...]) + pe


def _node_mlp_body(pn_ref, pp_ref, wi, bi, w1, b1, w2, b2, wo, bo, g, b,
                   out_ref):
    pn = pn_ref[...]
    pp = pp_ref[...]
    ps = pp[0] + pp[1]
    W = wi[...]
    h = _lrelu(_dot(pn, W[0:16]) + _dot(ps, W[16:32]) + bi[...])
    f = _mlp_tail(h, w1[...], b1[...], w2[...], b2[...], wo[...], bo[...])
    out_ref[...] = _ln(f, g[...], b[...]) + pn


def _decode_body(pn_ref, wi, bi, w1, b1, w2, b2, wo, bo, out_ref):
    h = _lrelu(_dot(pn_ref[...], wi[...]) + bi[...])
    out_ref[...] = _mlp_tail(h, w1[...], b1[...], w2[...], b2[...], wo[...],
                             bo[...])


@functools.lru_cache(maxsize=1)
def _sc_kernels():
    mesh = plsc.VectorSubcoreMesh(core_axis_name="c", subcore_axis_name="s")

    @functools.partial(
        pl.kernel,
        out_type=(jax.ShapeDtypeStruct((EPAD, 16), jnp.float32),
                  jax.ShapeDtypeStruct((EPAD, 16), jnp.float32)),
        mesh=mesh,
        scratch_types=[
            pltpu.VMEM((GSZ_G, CHUNK), jnp.int32),
            pltpu.VMEM((GRP_EG, 16), jnp.float32),
            pltpu.SemaphoreType.DMA,
        ],
        compiler_params=pltpu.CompilerParams(use_tc_tiling_on_sc=False),
    )
    def _gather_pair(pn_hbm, src_hbm, dst_hbm, osrc_hbm, odst_hbm,
                     idx_v, rows_v, sem):
        wid = lax.axis_index("c") * 16 + lax.axis_index("s")

        def one(iref, oref):
            def grp(gi, carry):
                crb = wid * PT_CH + gi * GSZ_G
                ebase = wid * PT_E + gi * GRP_EG
                pltpu.sync_copy(iref.at[pl.ds(crb, GSZ_G)], idx_v)
                cps = [pltpu.async_copy(pn_hbm.at[idx_v.at[j]],
                                        rows_v.at[pl.ds(j * CHUNK, CHUNK)],
                                        sem)
                       for j in range(GSZ_G)]
                for cp in cps:
                    cp.wait()
                pltpu.sync_copy(rows_v, oref.at[pl.ds(ebase, GRP_EG)])
                return carry
            lax.fori_loop(0, GROUPS_G, grp, 0)

        one(src_hbm, osrc_hbm)
        one(dst_hbm, odst_hbm)

    @functools.partial(
        pl.kernel,
        out_type=jax.ShapeDtypeStruct((2, N, 16), jnp.float32),
        mesh=mesh,
        scratch_types=[
            pltpu.VMEM((CHUNK, 16), jnp.float32),
            pltpu.VMEM((GSZ_S, CHUNK), jnp.int32),
            pltpu.VMEM((GRP_ES, 16), jnp.float32),
            pltpu.VMEM_SHARED((N_ACC, 16), jnp.float32),
            pltpu.SemaphoreType.DMA,
        ],
        compiler_params=pltpu.CompilerParams(use_tc_tiling_on_sc=False),
    )
    def _scatter_sum(rows_hbm, dst_hbm, out_hbm, zbuf, idx_v, rows_v, acc,
                     sem):
        c = lax.axis_index("c")
        s = lax.axis_index("s")
        wid = c * 16 + s

        def zrow(i, carry):
            zbuf[i, :] = jnp.zeros((16,), jnp.float32)
            return carry
        lax.fori_loop(0, CHUNK, zrow, 0)

        def zch(j, carry):
            pltpu.sync_copy(zbuf, acc.at[pl.ds(s * ZROWS + j * CHUNK, CHUNK)])
            return carry
        lax.fori_loop(0, ZCH, zch, 0)
        plsc.subcore_barrier()

        def grp(gi, carry):
            crb = wid * PT_CH + gi * GSZ_S
            ebase = wid * PT_E + gi * GRP_ES
            pltpu.sync_copy(dst_hbm.at[pl.ds(crb, GSZ_S)], idx_v)
            pltpu.sync_copy(rows_hbm.at[pl.ds(ebase, GRP_ES)], rows_v)
            for j in range(GSZ_S):
                pltpu.sync_copy(rows_v.at[pl.ds(j * CHUNK, CHUNK)],
                                acc.at[idx_v.at[j]], add=True)
            return carry
        lax.fori_loop(0, GROUPS_S, grp, 0)
        plsc.subcore_barrier()

        @pl.when(s < 15)
        def _():
            pltpu.sync_copy(acc.at[pl.ds(s * RD, RD)],
                            out_hbm.at[c, pl.ds(s * RD, RD)])

        @pl.when(s == 15)
        def _():
            pltpu.sync_copy(acc.at[pl.ds(15 * RD, RD_LAST)],
                            out_hbm.at[c, pl.ds(15 * RD, RD_LAST)])

    return _gather_pair, _scatter_sum


def kernel(nfeatures, efeatures, next_flowrate, weights, edge_index,
           inlet_mask):
    f32 = jnp.float32
    src = edge_index[0].astype(jnp.int32)
    dst = edge_index[1].astype(jnp.int32)
    pad = EPAD - E
    pad_ar = jnp.arange(pad, dtype=jnp.int32)
    src2d = jnp.concatenate([src, pad_ar % N]).reshape(NCH_TOT, CHUNK)
    dst2d = jnp.concatenate([dst, N + pad_ar % (N_ACC - N)]) \
        .reshape(NCH_TOT, CHUNK)
    flow2 = next_flowrate.astype(f32).reshape(N, 1)
    mask2 = inlet_mask.astype(jnp.int32).reshape(N, 1)

    w_enc_n = _flat_mlp(weights['enc_n'], True)
    w_enc_e = _flat_mlp(weights['enc_e'], True)
    w_out = _flat_mlp(weights['out'], False)

    pn = pl.pallas_call(
        _enc_n_body,
        grid=(N // RN,),
        in_specs=[pl.BlockSpec((RN, 11), lambda i: (i, 0)),
                  pl.BlockSpec((RN, 1), lambda i: (i, 0)),
                  pl.BlockSpec((RN, 1), lambda i: (i, 0)),
                  *[_wspec(w) for w in w_enc_n]],
        out_specs=pl.BlockSpec((RN, 16), lambda i: (i, 0)),
        out_shape=jax.ShapeDtypeStruct((N, 16), f32),
    )(nfeatures, flow2, mask2, *w_enc_n)

    ef_p = jnp.pad(efeatures.astype(f32), ((0, EPAD - E), (0, 0)))
    pe = pl.pallas_call(
        _enc_e_body,
        grid=(EPAD // RE,),
        in_specs=[pl.BlockSpec((RE, 4), lambda i: (i, 0)),
                  *[_wspec(w) for w in w_enc_e]],
        out_specs=pl.BlockSpec((RE, 16), lambda i: (i, 0)),
        out_shape=jax.ShapeDtypeStruct((EPAD, 16), f32),
    )(ef_p, *w_enc_e)

    for i in range(2):
        w_pe = _flat_mlp(weights['proc_e'][i], True)
        w_pnw = _flat_mlp(weights['proc_n'][i], True)

        gather_pair, scatter_sum = _sc_kernels()
        gs, gd = gather_pair(pn, src2d, dst2d)

        pe = pl.pallas_call(
            _edge_mlp_body,
            grid=(EPAD // RE,),
            in_specs=[pl.BlockSpec((RE, 16), lambda i: (i, 0)),
                      pl.BlockSpec((RE, 16), lambda i: (i, 0)),
                      pl.BlockSpec((RE, 16), lambda i: (i, 0)),
                      *[_wspec(w) for w in w_pe]],
            out_specs=pl.BlockSpec((RE, 16), lambda i: (i, 0)),
            out_shape=jax.ShapeDtypeStruct((EPAD, 16), f32),
        )(pe, gs, gd, *w_pe)

        pp = scatter_sum(pe, dst2d)

        pn = pl.pallas_call(
            _node_mlp_body,
            grid=(N // RN,),
            in_specs=[pl.BlockSpec((RN, 16), lambda i: (i, 0)),
                      pl.BlockSpec((2, RN, 16), lambda i: (0, i, 0)),
                      *[_wspec(w) for w in w_pnw]],
            out_specs=pl.BlockSpec((RN, 16), lambda i: (i, 0)),
            out_shape=jax.ShapeDtypeStruct((N, 16), f32),
        )(pn, pp, *w_pnw)

    pred = pl.pallas_call(
        _decode_body,
        grid=(N // RN,),
        in_specs=[pl.BlockSpec((RN, 16), lambda i: (i, 0)),
                  *[_wspec(w) for w in w_out]],
        out_specs=pl.BlockSpec((RN, 2), lambda i: (i, 0)),
        out_shape=jax.ShapeDtypeStruct((N, 2), f32),
    )(pn, *w_out)
    return pred
